# baseline (speedup 1.0000x reference)
import functools

import jax
import jax.numpy as jnp
from jax import lax
from jax.experimental import pallas as pl
from jax.experimental.pallas import tpu as pltpu
from jax.experimental.pallas import tpu_sc as plsc

LANES = 16


def _sqrt_f32(s):
  i = lax.bitcast_convert_type(s, jnp.int32)
  i = jnp.int32(0x5F3759DF) - lax.shift_right_logical(i, 1)
  y = lax.bitcast_convert_type(i, jnp.float32)
  h = s * jnp.float32(0.5)
  y = y * (jnp.float32(1.5) - h * y * y)
  y = y * (jnp.float32(1.5) - h * y * y)
  y = y * (jnp.float32(1.5) - h * y * y)
  return s * y


@functools.partial(jax.jit, static_argnames=("n_atoms", "row", "n_geoms", "n_edges",
                                             "batch", "nbatch", "per_worker"))
def _run(x2, bi, bj, *, n_atoms, row, n_geoms, n_edges, batch, nbatch, per_worker):
  info = plsc.get_sparse_core_info()
  nc, ns = info.num_cores, info.num_subcores
  gv = n_geoms // LANES

  mesh = plsc.VectorSubcoreMesh(core_axis_name="c", subcore_axis_name="s")

  @functools.partial(
      pl.kernel,
      mesh=mesh,
      out_type=jax.ShapeDtypeStruct((n_edges, n_geoms), jnp.float32),
      scratch_types=[
          pltpu.VMEM((batch,), jnp.int32),
          pltpu.VMEM((batch,), jnp.int32),
          pltpu.VMEM((batch, row), jnp.float32),
          pltpu.VMEM((batch, row), jnp.float32),
          pltpu.VMEM((batch, n_geoms), jnp.float32),
          pltpu.SemaphoreType.DMA,
          pltpu.SemaphoreType.DMA,
      ],
  )
  def k(x_hbm, bi_hbm, bj_hbm, out_hbm, idxi_v, idxj_v, ri_v, rj_v, out_v,
        semi, semj):
    wid = lax.axis_index("s") * nc + lax.axis_index("c")
    wbase = wid * per_worker

    def batch_body(b, _):
      base = wbase + b * batch

      @pl.when(base < n_edges)
      def _():
        pltpu.sync_copy(bi_hbm.at[pl.ds(base, batch)], idxi_v)
        pltpu.sync_copy(bj_hbm.at[pl.ds(base, batch)], idxj_v)
        cpi = pltpu.make_async_copy(x_hbm.at[idxi_v], ri_v, semi)
        cpj = pltpu.make_async_copy(x_hbm.at[idxj_v], rj_v, semj)
        cpi.start()
        cpj.start()
        cpi.wait()
        cpj.wait()

        def edge_body(e, _):
          for g in range(gv):
            off = g * LANES
            d0 = ri_v[e, pl.ds(off, LANES)] - rj_v[e, pl.ds(off, LANES)]
            d1 = (ri_v[e, pl.ds(n_geoms + off, LANES)]
                  - rj_v[e, pl.ds(n_geoms + off, LANES)])
            d2 = (ri_v[e, pl.ds(2 * n_geoms + off, LANES)]
                  - rj_v[e, pl.ds(2 * n_geoms + off, LANES)])
            s = d0 * d0 + d1 * d1 + d2 * d2
            out_v[e, pl.ds(off, LANES)] = _sqrt_f32(s)
          return 0

        lax.fori_loop(0, batch, edge_body, 0, unroll=False)

        full = base + batch <= n_edges

        @pl.when(full)
        def _():
          pltpu.sync_copy(out_v, out_hbm.at[pl.ds(base, batch), :])

        @pl.when(jnp.logical_not(full))
        def _():
          def tail_body(e, _):
            @pl.when(base + e < n_edges)
            def _():
              pltpu.sync_copy(out_v.at[e], out_hbm.at[base + e])
            return 0

          lax.fori_loop(0, batch, tail_body, 0, unroll=False)

      return 0

    lax.fori_loop(0, nbatch, batch_body, 0, unroll=False)

  return k(x2, bi, bj)


def kernel(input, bonds):
  n_atoms, three, n_geoms = input.shape
  n_edges = bonds.shape[0]
  row = three * n_geoms

  batch = 16
  n_workers = 32
  per_worker_edges = -(-n_edges // n_workers)
  nbatch = -(-per_worker_edges // batch)
  per_worker = nbatch * batch
  e_pad = n_workers * per_worker

  x2 = input.reshape(n_atoms, row)
  bpad = jnp.concatenate(
      [bonds, jnp.zeros((e_pad - n_edges, 2), jnp.int32)], axis=0)
  bi = bpad[:, 0]
  bj = bpad[:, 1]

  return _run(x2, bi, bj, n_atoms=n_atoms, row=row, n_geoms=n_geoms,
              n_edges=n_edges, batch=batch, nbatch=nbatch,
              per_worker=per_worker)


# speedup vs baseline: 1.4665x; 1.4665x over previous
import functools

import jax
import jax.numpy as jnp
from jax import lax
from jax.experimental import pallas as pl
from jax.experimental.pallas import tpu as pltpu
from jax.experimental.pallas import tpu_sc as plsc

LANES = 16
BATCH = 16


def _sqrt_f32(s):
  i = lax.bitcast_convert_type(s, jnp.int32)
  i = jnp.int32(0x5F3759DF) - lax.shift_right_logical(i, 1)
  y = lax.bitcast_convert_type(i, jnp.float32)
  h = s * jnp.float32(0.5)
  y = y * (jnp.float32(1.5) - h * y * y)
  y = y * (jnp.float32(1.5) - h * y * y)
  return s * y


@functools.partial(jax.jit, static_argnames=("n_atoms", "row", "n_geoms",
                                             "n_edges", "nbatch", "per_worker"))
def _run(x2, bflat, *, n_atoms, row, n_geoms, n_edges, nbatch, per_worker):
  info = plsc.get_sparse_core_info()
  nc = info.num_cores
  gv = n_geoms // LANES
  nhalf = nbatch // 2

  mesh = plsc.VectorSubcoreMesh(core_axis_name="c", subcore_axis_name="s")

  @functools.partial(
      pl.kernel,
      mesh=mesh,
      out_type=jax.ShapeDtypeStruct((n_edges, n_geoms), jnp.float32),
      scratch_types=[
          pltpu.VMEM((2 * per_worker,), jnp.int32),
          pltpu.VMEM((2 * BATCH, row), jnp.float32),
          pltpu.VMEM((2 * BATCH, row), jnp.float32),
          pltpu.VMEM((BATCH, n_geoms), jnp.float32),
          pltpu.VMEM((BATCH, n_geoms), jnp.float32),
          pltpu.SemaphoreType.DMA,
          pltpu.SemaphoreType.DMA,
          pltpu.SemaphoreType.DMA,
          pltpu.SemaphoreType.DMA,
      ],
  )
  def k(x_hbm, b_hbm, out_hbm, idx_all, rows0, rows1, out0, out1,
        gsem0, gsem1, osem0, osem1):
    rows = (rows0, rows1)
    outs = (out0, out1)
    gsems = (gsem0, gsem1)
    osems = (osem0, osem1)

    wid = lax.axis_index("s") * nc + lax.axis_index("c")
    wbase = wid * per_worker

    def gather_cp(kb, sl):
      off = pl.multiple_of(kb * (2 * BATCH), 2 * BATCH)
      idx_sl = idx_all.at[pl.ds(off, 2 * BATCH)]
      return pltpu.make_async_copy(x_hbm.at[idx_sl], rows[sl], gsems[sl])

    def store_cp(base, sl):
      return pltpu.make_async_copy(
          outs[sl], out_hbm.at[pl.ds(base, BATCH), :], osems[sl])

    def compute_batch(rv, ov):
      def edge_body(e, _):
        ei = 2 * e
        ej = 2 * e + 1
        for g in range(gv):
          off = g * LANES
          d0 = rv[ei, pl.ds(off, LANES)] - rv[ej, pl.ds(off, LANES)]
          d1 = (rv[ei, pl.ds(n_geoms + off, LANES)]
                - rv[ej, pl.ds(n_geoms + off, LANES)])
          d2 = (rv[ei, pl.ds(2 * n_geoms + off, LANES)]
                - rv[ej, pl.ds(2 * n_geoms + off, LANES)])
          s = d0 * d0 + d1 * d1 + d2 * d2
          ov[e, pl.ds(off, LANES)] = _sqrt_f32(s)
        return 0

      lax.fori_loop(0, BATCH, edge_body, 0, unroll=False)

    any_live = wbase < n_edges

    @pl.when(any_live)
    def _():
      pltpu.sync_copy(b_hbm.at[pl.ds(2 * wbase, 2 * per_worker)], idx_all)
      gather_cp(0, 0).start()

    def pair_body(kk, _):
      for sub in (0, 1):
        kb = 2 * kk + sub
        base = wbase + kb * BATCH
        live = base < n_edges

        @pl.when(live)
        def _(kb=kb, base=base, sl=sub, kk=kk):
          gather_cp(kb, sl).wait()

          next_ok = (wbase + (kb + 1) * BATCH) < n_edges
          if sl == 1:
            next_ok = jnp.logical_and(next_ok, kk < nhalf - 1)

          @pl.when(next_ok)
          def _():
            gather_cp(kb + 1, 1 - sl).start()

          @pl.when(kb >= 2)
          def _():
            store_cp(base - 2 * BATCH, sl).wait()

          compute_batch(rows[sl], outs[sl])

          full = base + BATCH <= n_edges

          @pl.when(full)
          def _():
            store_cp(base, sl).start()

          @pl.when(jnp.logical_not(full))
          def _():
            def tail_body(e, _):
              @pl.when(base + e < n_edges)
              def _():
                pltpu.sync_copy(outs[sl].at[e], out_hbm.at[base + e])
              return 0

            lax.fori_loop(0, BATCH, tail_body, 0, unroll=False)

      return 0

    lax.fori_loop(0, nhalf, pair_body, 0, unroll=False)

    n_my = jnp.maximum(jnp.int32(0),
                       jnp.minimum(jnp.int32(n_edges) - wbase,
                                   jnp.int32(per_worker)))
    n_live = (n_my + BATCH - 1) // BATCH
    n_full = n_my // BATCH
    for kq in (2, 1):
      kp = n_live - kq

      @pl.when(jnp.logical_and(kp >= 0, kp < n_full))
      def _(kp=kp):
        sl = lax.rem(kp, jnp.int32(2))

        @pl.when(sl == 0)
        def _():
          store_cp(wbase + kp * BATCH, 0).wait()

        @pl.when(sl == 1)
        def _():
          store_cp(wbase + kp * BATCH, 1).wait()

  return k(x2, bflat)


def kernel(input, bonds):
  n_atoms, three, n_geoms = input.shape
  n_edges = bonds.shape[0]
  row = three * n_geoms

  n_workers = 32
  per_worker_edges = -(-n_edges // n_workers)
  nbatch = -(-per_worker_edges // BATCH)
  nbatch += nbatch % 2
  per_worker = nbatch * BATCH
  e_pad = n_workers * per_worker

  x2 = input.reshape(n_atoms, row)
  bflat = jnp.concatenate(
      [bonds.reshape(-1), jnp.zeros(2 * (e_pad - n_edges), jnp.int32)])

  return _run(x2, bflat, n_atoms=n_atoms, row=row, n_geoms=n_geoms,
              n_edges=n_edges, nbatch=nbatch, per_worker=per_worker)


# speedup vs baseline: 3.9652x; 2.7038x over previous
import functools

import jax
import jax.numpy as jnp
from jax import lax
from jax.experimental import pallas as pl
from jax.experimental.pallas import tpu as pltpu
from jax.experimental.pallas import tpu_sc as plsc

LANES = 16
BATCH = 16


def _sqrt_group(ss):
  iv = [lax.bitcast_convert_type(s, jnp.int32) for s in ss]
  iv = [jnp.int32(0x5F3759DF) - lax.shift_right_logical(i, 1) for i in iv]
  ys = [lax.bitcast_convert_type(i, jnp.float32) for i in iv]
  hs = [s * jnp.float32(0.5) for s in ss]
  for _ in range(2):
    ts = [y * y for y in ys]
    ts = [h * t for h, t in zip(hs, ts)]
    ts = [jnp.float32(1.5) - t for t in ts]
    ys = [y * t for y, t in zip(ys, ts)]
  return [s * y for s, y in zip(ss, ys)]


@functools.partial(jax.jit, static_argnames=("n_atoms", "row", "n_geoms",
                                             "n_edges", "nbatch", "per_worker"))
def _run(x2, bflat, *, n_atoms, row, n_geoms, n_edges, nbatch, per_worker):
  info = plsc.get_sparse_core_info()
  nc = info.num_cores
  gv = n_geoms // LANES
  nhalf = nbatch // 2

  mesh = plsc.VectorSubcoreMesh(core_axis_name="c", subcore_axis_name="s")

  @functools.partial(
      pl.kernel,
      mesh=mesh,
      out_type=jax.ShapeDtypeStruct((n_edges, n_geoms), jnp.float32),
      scratch_types=[
          pltpu.VMEM((2 * per_worker,), jnp.int32),
          pltpu.VMEM((2 * BATCH, row), jnp.float32),
          pltpu.VMEM((2 * BATCH, row), jnp.float32),
          pltpu.VMEM((BATCH, n_geoms), jnp.float32),
          pltpu.VMEM((BATCH, n_geoms), jnp.float32),
          pltpu.SemaphoreType.DMA,
          pltpu.SemaphoreType.DMA,
          pltpu.SemaphoreType.DMA,
          pltpu.SemaphoreType.DMA,
      ],
  )
  def k(x_hbm, b_hbm, out_hbm, idx_all, rows0, rows1, out0, out1,
        gsem0, gsem1, osem0, osem1):
    rows = (rows0, rows1)
    outs = (out0, out1)
    gsems = (gsem0, gsem1)
    osems = (osem0, osem1)

    wid = lax.axis_index("s") * nc + lax.axis_index("c")
    wbase = wid * per_worker

    def gather_cp(kb, sl):
      off = pl.multiple_of(kb * (2 * BATCH), 2 * BATCH)
      idx_sl = idx_all.at[pl.ds(off, 2 * BATCH)]
      return pltpu.make_async_copy(x_hbm.at[idx_sl], rows[sl], gsems[sl])

    def store_cp(base, sl):
      return pltpu.make_async_copy(
          outs[sl], out_hbm.at[pl.ds(base, BATCH), :], osems[sl])

    def compute_batch(rv, ov):
      group = 8

      def edge_body(e, _):
        ei = 2 * e
        ej = 2 * e + 1
        for g0 in range(0, gv, group):
          ss = []
          for g in range(g0, g0 + group):
            off = g * LANES
            d0 = rv[ei, pl.ds(off, LANES)] - rv[ej, pl.ds(off, LANES)]
            d1 = (rv[ei, pl.ds(n_geoms + off, LANES)]
                  - rv[ej, pl.ds(n_geoms + off, LANES)])
            d2 = (rv[ei, pl.ds(2 * n_geoms + off, LANES)]
                  - rv[ej, pl.ds(2 * n_geoms + off, LANES)])
            ss.append(d0 * d0 + d1 * d1 + d2 * d2)
          rr = _sqrt_group(ss)
          for g, r in zip(range(g0, g0 + group), rr):
            ov[e, pl.ds(g * LANES, LANES)] = r
        return 0

      lax.fori_loop(0, BATCH, edge_body, 0, unroll=False)

    any_live = wbase < n_edges

    @pl.when(any_live)
    def _():
      pltpu.sync_copy(b_hbm.at[pl.ds(2 * wbase, 2 * per_worker)], idx_all)
      gather_cp(0, 0).start()

    def pair_body(kk, _):
      for sub in (0, 1):
        kb = 2 * kk + sub
        base = wbase + kb * BATCH
        live = base < n_edges

        @pl.when(live)
        def _(kb=kb, base=base, sl=sub, kk=kk):
          gather_cp(kb, sl).wait()

          next_ok = (wbase + (kb + 1) * BATCH) < n_edges
          if sl == 1:
            next_ok = jnp.logical_and(next_ok, kk < nhalf - 1)

          @pl.when(next_ok)
          def _():
            gather_cp(kb + 1, 1 - sl).start()

          @pl.when(kb >= 2)
          def _():
            store_cp(base - 2 * BATCH, sl).wait()

          compute_batch(rows[sl], outs[sl])

          full = base + BATCH <= n_edges

          @pl.when(full)
          def _():
            store_cp(base, sl).start()

          @pl.when(jnp.logical_not(full))
          def _():
            def tail_body(e, _):
              @pl.when(base + e < n_edges)
              def _():
                pltpu.sync_copy(outs[sl].at[e], out_hbm.at[base + e])
              return 0

            lax.fori_loop(0, BATCH, tail_body, 0, unroll=False)

      return 0

    lax.fori_loop(0, nhalf, pair_body, 0, unroll=False)

    n_my = jnp.maximum(jnp.int32(0),
                       jnp.minimum(jnp.int32(n_edges) - wbase,
                                   jnp.int32(per_worker)))
    n_live = (n_my + BATCH - 1) // BATCH
    n_full = n_my // BATCH
    for kq in (2, 1):
      kp = n_live - kq

      @pl.when(jnp.logical_and(kp >= 0, kp < n_full))
      def _(kp=kp):
        sl = lax.rem(kp, jnp.int32(2))

        @pl.when(sl == 0)
        def _():
          store_cp(wbase + kp * BATCH, 0).wait()

        @pl.when(sl == 1)
        def _():
          store_cp(wbase + kp * BATCH, 1).wait()

  return k(x2, bflat)


def kernel(input, bonds):
  n_atoms, three, n_geoms = input.shape
  n_edges = bonds.shape[0]
  row = three * n_geoms

  n_workers = 32
  per_worker_edges = -(-n_edges // n_workers)
  nbatch = -(-per_worker_edges // BATCH)
  nbatch += nbatch % 2
  per_worker = nbatch * BATCH
  e_pad = n_workers * per_worker

  x2 = input.reshape(n_atoms, row)
  bflat = jnp.concatenate(
      [bonds.reshape(-1), jnp.zeros(2 * (e_pad - n_edges), jnp.int32)])

  return _run(x2, bflat, n_atoms=n_atoms, row=row, n_geoms=n_geoms,
              n_edges=n_edges, nbatch=nbatch, per_worker=per_worker)


# speedup vs baseline: 4.8314x; 1.2185x over previous
import functools

import jax
import jax.numpy as jnp
from jax import lax
from jax.experimental import pallas as pl
from jax.experimental.pallas import tpu as pltpu
from jax.experimental.pallas import tpu_sc as plsc

LANES = 16
BATCH = 16


def _sqrt_group(ss):
  iv = [lax.bitcast_convert_type(s, jnp.int32) for s in ss]
  iv = [jnp.int32(0x5F3759DF) - lax.shift_right_logical(i, 1) for i in iv]
  ys = [lax.bitcast_convert_type(i, jnp.float32) for i in iv]
  hs = [s * jnp.float32(0.5) for s in ss]
  for _ in range(2):
    ts = [y * y for y in ys]
    ts = [h * t for h, t in zip(hs, ts)]
    ts = [jnp.float32(1.5) - t for t in ts]
    ys = [y * t for y, t in zip(ys, ts)]
  return [s * y for s, y in zip(ss, ys)]


@functools.partial(jax.jit, static_argnames=("n_atoms", "row", "n_geoms",
                                             "n_edges", "nbatch", "per_worker"))
def _run(x2, bflat, *, n_atoms, row, n_geoms, n_edges, nbatch, per_worker):
  info = plsc.get_sparse_core_info()
  nc = info.num_cores
  gv = n_geoms // LANES
  nhalf = nbatch // 2

  mesh = plsc.VectorSubcoreMesh(core_axis_name="c", subcore_axis_name="s")

  @functools.partial(
      pl.kernel,
      mesh=mesh,
      out_type=jax.ShapeDtypeStruct((n_edges, n_geoms), jnp.float32),
      scratch_types=[
          pltpu.VMEM((2 * per_worker,), jnp.int32),
          pltpu.VMEM((2 * BATCH, row), jnp.float32),
          pltpu.VMEM((2 * BATCH, row), jnp.float32),
          pltpu.VMEM((BATCH, n_geoms), jnp.float32),
          pltpu.VMEM((BATCH, n_geoms), jnp.float32),
          pltpu.SemaphoreType.DMA,
          pltpu.SemaphoreType.DMA,
          pltpu.SemaphoreType.DMA,
          pltpu.SemaphoreType.DMA,
      ],
  )
  def k(x_hbm, b_hbm, out_hbm, idx_all, rows0, rows1, out0, out1,
        gsem0, gsem1, osem0, osem1):
    rows = (rows0, rows1)
    outs = (out0, out1)
    gsems = (gsem0, gsem1)
    osems = (osem0, osem1)

    wid = lax.axis_index("s") * nc + lax.axis_index("c")
    wbase = wid * per_worker

    def gather_cp(kb, sl):
      off = pl.multiple_of(kb * (2 * BATCH), 2 * BATCH)
      idx_sl = idx_all.at[pl.ds(off, 2 * BATCH)]
      return pltpu.make_async_copy(x_hbm.at[idx_sl], rows[sl], gsems[sl])

    def store_cp(base, sl):
      return pltpu.make_async_copy(
          outs[sl], out_hbm.at[pl.ds(base, BATCH), :], osems[sl])

    def compute_batch(rv, ov):
      group = 8

      def edge_body(e, _):
        ei = 2 * e
        ej = 2 * e + 1
        prev = None
        for g0 in range(0, gv, group):
          ss = []
          for g in range(g0, g0 + group):
            off = g * LANES
            d0 = rv[ei, pl.ds(off, LANES)] - rv[ej, pl.ds(off, LANES)]
            d1 = (rv[ei, pl.ds(n_geoms + off, LANES)]
                  - rv[ej, pl.ds(n_geoms + off, LANES)])
            d2 = (rv[ei, pl.ds(2 * n_geoms + off, LANES)]
                  - rv[ej, pl.ds(2 * n_geoms + off, LANES)])
            ss.append(d0 * d0 + d1 * d1 + d2 * d2)
          if prev is not None:
            pg, pr = prev
            for g, r in zip(range(pg, pg + group), pr):
              ov[e, pl.ds(g * LANES, LANES)] = r
          prev = (g0, _sqrt_group(ss))
        pg, pr = prev
        for g, r in zip(range(pg, pg + group), pr):
          ov[e, pl.ds(g * LANES, LANES)] = r
        return 0

      lax.fori_loop(0, BATCH, edge_body, 0, unroll=False)

    any_live = wbase < n_edges

    @pl.when(any_live)
    def _():
      pltpu.sync_copy(b_hbm.at[pl.ds(2 * wbase, 2 * per_worker)], idx_all)
      gather_cp(0, 0).start()

    def pair_body(kk, _):
      for sub in (0, 1):
        kb = 2 * kk + sub
        base = wbase + kb * BATCH
        live = base < n_edges

        @pl.when(live)
        def _(kb=kb, base=base, sl=sub, kk=kk):
          gather_cp(kb, sl).wait()

          next_ok = (wbase + (kb + 1) * BATCH) < n_edges
          if sl == 1:
            next_ok = jnp.logical_and(next_ok, kk < nhalf - 1)

          @pl.when(next_ok)
          def _():
            gather_cp(kb + 1, 1 - sl).start()

          @pl.when(kb >= 2)
          def _():
            store_cp(base - 2 * BATCH, sl).wait()

          compute_batch(rows[sl], outs[sl])

          full = base + BATCH <= n_edges

          @pl.when(full)
          def _():
            store_cp(base, sl).start()

          @pl.when(jnp.logical_not(full))
          def _():
            def tail_body(e, _):
              @pl.when(base + e < n_edges)
              def _():
                pltpu.sync_copy(outs[sl].at[e], out_hbm.at[base + e])
              return 0

            lax.fori_loop(0, BATCH, tail_body, 0, unroll=False)

      return 0

    lax.fori_loop(0, nhalf, pair_body, 0, unroll=False)

    n_my = jnp.maximum(jnp.int32(0),
                       jnp.minimum(jnp.int32(n_edges) - wbase,
                                   jnp.int32(per_worker)))
    n_live = (n_my + BATCH - 1) // BATCH
    n_full = n_my // BATCH
    for kq in (2, 1):
      kp = n_live - kq

      @pl.when(jnp.logical_and(kp >= 0, kp < n_full))
      def _(kp=kp):
        sl = lax.rem(kp, jnp.int32(2))

        @pl.when(sl == 0)
        def _():
          store_cp(wbase + kp * BATCH, 0).wait()

        @pl.when(sl == 1)
        def _():
          store_cp(wbase + kp * BATCH, 1).wait()

  return k(x2, bflat)


def kernel(input, bonds):
  n_atoms, three, n_geoms = input.shape
  n_edges = bonds.shape[0]
  row = three * n_geoms

  n_workers = 32
  per_worker_edges = -(-n_edges // n_workers)
  nbatch = -(-per_worker_edges // BATCH)
  nbatch += nbatch % 2
  per_worker = nbatch * BATCH
  e_pad = n_workers * per_worker

  x2 = input.reshape(n_atoms, row)
  bflat = jnp.concatenate(
      [bonds.reshape(-1), jnp.zeros(2 * (e_pad - n_edges), jnp.int32)])

  return _run(x2, bflat, n_atoms=n_atoms, row=row, n_geoms=n_geoms,
              n_edges=n_edges, nbatch=nbatch, per_worker=per_worker)
